# baseline (speedup 1.0000x reference)
import functools

import jax
import jax.numpy as jnp
from jax import lax
from jax.experimental import pallas as pl
from jax.experimental.pallas import tpu as pltpu
from jax.experimental.pallas import tpu_sc as plsc

B, L, D = 1024, 512, 64
DP = 128
NC, NS = 2, 16
NW = NC * NS
ROWS_PER_W = B // NW
NQ = 4
QL = L // NQ
LANES = 16
NG = QL // LANES


def _body(x_hbm, tok_hbm, pos_hbm, out_hbm,
          idx_v, idx2_v, off_v, bufa, bufb, outa, outb, pos_v, sem_g, sem_w):
    c = lax.axis_index("c")
    s = lax.axis_index("s")
    wid = s * NC + c
    base = wid * ROWS_PER_W

    pltpu.sync_copy(pos_hbm, pos_v)
    lane = jnp.arange(LANES, dtype=jnp.int32)

    def compute_quarter(q, buf, outq):
        def per_group(g):
            tloc = g * LANES + lane
            off16 = off_v[pl.ds(q * QL + g * LANES, LANES)]

            def per_feature(f, _):
                col = off16 + f
                v = plsc.load_gather(buf, [tloc, col])
                p = pos_v[f, pl.ds(q * QL + g * LANES, LANES)]
                outq[f, pl.ds(g * LANES, LANES)] = v + p
                return 0

            lax.fori_loop(0, D, per_feature, 0)

        for g in range(NG):
            per_group(g)

    def do_row(r, _):
        row = base + r
        pltpu.sync_copy(x_hbm.at[row], idx_v)
        for i in range(L // LANES):
            sl = pl.ds(i * LANES, LANES)
            v = idx_v[sl]
            idx2_v[sl] = lax.shift_right_logical(v, 1)
            off_v[sl] = (v & 1) * D

        def gather(q, buf):
            return pltpu.async_copy(
                tok_hbm.at[idx2_v.at[pl.ds(q * QL, QL)]], buf, sem_g
            )

        def store(q, outq):
            return pltpu.async_copy(
                outq, out_hbm.at[row, :, pl.ds(q * QL, QL)], sem_w
            )

        g0 = gather(0, bufa)
        g1 = gather(1, bufb)
        g0.wait()
        compute_quarter(0, bufa, outa)
        w0 = store(0, outa)
        g2 = gather(2, bufa)
        g1.wait()
        compute_quarter(1, bufb, outb)
        w1 = store(1, outb)
        g3 = gather(3, bufb)
        g2.wait()
        w0.wait()
        compute_quarter(2, bufa, outa)
        w2 = store(2, outa)
        g3.wait()
        w1.wait()
        compute_quarter(3, bufb, outb)
        w3 = store(3, outb)
        w2.wait()
        w3.wait()
        return 0

    lax.fori_loop(0, ROWS_PER_W, do_row, 0)


_emb = functools.partial(
    pl.kernel,
    out_type=jax.ShapeDtypeStruct((B, D, L), jnp.float32),
    mesh=plsc.VectorSubcoreMesh(core_axis_name="c", subcore_axis_name="s"),
    scratch_types=[
        pltpu.VMEM((L,), jnp.int32),
        pltpu.VMEM((L,), jnp.int32),
        pltpu.VMEM((L,), jnp.int32),
        pltpu.VMEM((QL, DP), jnp.float32),
        pltpu.VMEM((QL, DP), jnp.float32),
        pltpu.VMEM((D, QL), jnp.float32),
        pltpu.VMEM((D, QL), jnp.float32),
        pltpu.VMEM((D, L), jnp.float32),
        pltpu.SemaphoreType.DMA,
        pltpu.SemaphoreType.DMA,
    ],
    compiler_params=pltpu.CompilerParams(
        use_tc_tiling_on_sc=True, needs_layout_passes=False
    ),
)(_body)


@jax.jit
def kernel(x, token_emb, pos_emb):
    tok2 = token_emb.reshape(500000, DP)
    pos_t = pos_emb.T
    out_t = _emb(x.astype(jnp.int32), tok2, pos_t)
    return jnp.transpose(out_t, (0, 2, 1))
